# baseline (speedup 1.0000x reference)
import functools

import jax
import jax.numpy as jnp
from jax import lax
from jax.experimental import pallas as pl
from jax.experimental.pallas import tpu as pltpu
from jax.experimental.pallas import tpu_sc as plsc

NUM_CORES = 2
NUM_SUBCORES = 16
NUM_WORKERS = NUM_CORES * NUM_SUBCORES
LANES = 16


def _make_spmm(n_nodes: int, n_edges: int, d: int):
    epw = n_edges // NUM_WORKERS
    K = 80
    ch = epw // K
    assert epw * NUM_WORKERS == n_edges and ch * K == epw
    nblk = n_nodes // K
    assert nblk * K == n_nodes
    bps = -(-nblk // NUM_SUBCORES)
    nvec = d // LANES

    mesh = plsc.VectorSubcoreMesh(core_axis_name="c", subcore_axis_name="s")

    NB = 3
    UNROLL = 8
    assert K % UNROLL == 0
    assert ch >= 5 and (ch - 2) % NB == 0

    @functools.partial(
        pl.kernel,
        out_type=jax.ShapeDtypeStruct((NUM_CORES, n_nodes, d), jnp.float32),
        mesh=mesh,
        scratch_types=(
            [pltpu.VMEM((K,), jnp.int32) for _ in range(NB)]
            + [pltpu.VMEM((K,), jnp.int32) for _ in range(NB)]
            + [pltpu.VMEM((K,), jnp.float32) for _ in range(NB)]
            + [pltpu.VMEM((K, d), jnp.float32) for _ in range(NB)]
            + [pltpu.VMEM_SHARED((n_nodes, d), jnp.float32)]
            + [pltpu.SemaphoreType.DMA for _ in range(3 * NB)]
        ),
        compiler_params=pltpu.CompilerParams(needs_layout_passes=False),
    )
    def spmm(ei_hbm, w_hbm, h_hbm, out_hbm,
             sv0, sv1, sv2, dv0, dv1, dv2, wv0, wv1, wv2, rv0, rv1, rv2,
             acc, as0, as1, as2, gs0, gs1, gs2, ss0, ss1, ss2):
        src_v, dst_v, w_v = (sv0, sv1, sv2), (dv0, dv1, dv2), (wv0, wv1, wv2)
        rows_v = (rv0, rv1, rv2)
        asem, gsem, ssem = (as0, as1, as2), (gs0, gs1, gs2), (ss0, ss1, ss2)

        c = lax.axis_index("c")
        s = lax.axis_index("s")
        wid = c * NUM_SUBCORES + s
        ebase = wid * epw

        def stage(i, b):
            ebeg = ebase + i * K
            pltpu.async_copy(ei_hbm.at[pl.ds(ebeg, K)], src_v[b], asem[b])
            pltpu.async_copy(
                ei_hbm.at[pl.ds(n_edges + ebeg, K)], dst_v[b], asem[b])
            pltpu.async_copy(w_hbm.at[pl.ds(ebeg, K)], w_v[b], asem[b])

        def wait_stage(b):
            dsl = pl.ds(0, K)
            pltpu.make_async_copy(ei_hbm.at[dsl], src_v[b], asem[b]).wait()
            pltpu.make_async_copy(ei_hbm.at[dsl], dst_v[b], asem[b]).wait()
            pltpu.make_async_copy(w_hbm.at[dsl], w_v[b], asem[b]).wait()

        def gather(b):
            pltpu.async_copy(h_hbm.at[src_v[b]], rows_v[b], gsem[b])

        def wait_gather(b):
            pltpu.make_async_copy(
                h_hbm.at[pl.ds(0, K)], rows_v[b], gsem[b]).wait()

        def scale(b):
            @plsc.parallel_loop(0, K, 1, unroll=UNROLL)
            def _(j):
                wvec = plsc.load_gather(
                    w_v[b], [jnp.full((LANES,), j, jnp.int32)])
                packs = [
                    plsc.bitcast(rows_v[b][j, pl.ds(v * LANES, LANES)],
                                 jnp.bfloat16)
                    for v in range(nvec // 2)
                ]
                pairs = [plsc.unpack(p, format=plsc.PackFormat.INTERLEAVED)
                         for p in packs]
                for v, (lo, hi) in enumerate(pairs):
                    rows_v[b][j, pl.ds(v * LANES, LANES)] = lo * wvec
                for v, (lo, hi) in enumerate(pairs):
                    rows_v[b][j, pl.ds(d // 2 + v * LANES, LANES)] = hi * wvec

        def scatter(b):
            pltpu.async_copy(rows_v[b], acc.at[dst_v[b]], ssem[b], add=True)

        def wait_scatter(b):
            pltpu.make_async_copy(
                rows_v[b], acc.at[pl.ds(0, K)], ssem[b]).wait()

        def zrow(r, carry):
            for v in range(nvec):
                rv0[r, pl.ds(v * LANES, LANES)] = jnp.zeros(
                    (LANES,), jnp.float32)
            return carry
        lax.fori_loop(0, K, zrow, 0)
        for k in range(bps):
            blk = s + NUM_SUBCORES * k

            @pl.when(blk < nblk)
            def _():
                pltpu.sync_copy(rv0, acc.at[pl.ds(blk * K, K)])
        plsc.subcore_barrier()

        stage(0, 0)
        stage(1, 1)
        wait_stage(0)
        gather(0)
        wait_stage(1)
        gather(1)
        wait_gather(0)
        scale(0)
        scatter(0)
        stage(2, 2)
        wait_stage(2)
        gather(2)
        wait_gather(1)
        scale(1)
        scatter(1)
        wait_scatter(0)
        stage(3, 0)

        def body(t, carry):
            i0 = 3 * t + 2
            for k in range(3):
                ph = (2 + k) % 3
                nxt = (ph + 1) % 3
                prv = (ph + 2) % 3
                i = i0 + k
                wait_stage(nxt)
                gather(nxt)
                wait_gather(ph)
                scale(ph)
                scatter(ph)
                wait_scatter(prv)
                stage(i + 2, prv)
            return carry
        lax.fori_loop(0, (ch - 5) // 3, body, 0)

        wait_stage(0)
        gather(0)
        wait_gather(2)
        scale(2)
        scatter(2)
        wait_scatter(1)
        stage(ch - 1, 1)
        wait_stage(1)
        gather(1)
        wait_gather(0)
        scale(0)
        scatter(0)
        wait_gather(1)
        scale(1)
        scatter(1)
        wait_scatter(2)
        wait_scatter(0)
        wait_scatter(1)
        plsc.subcore_barrier()

        for k in range(bps):
            blk = s + NUM_SUBCORES * k

            @pl.when(blk < nblk)
            def _():
                sl = pl.ds(blk * K, K)
                pltpu.sync_copy(acc.at[sl], out_hbm.at[c, sl])

    return spmm


def _pack_cols(v):
    m = v.shape[1] // 2
    lo = lax.bitcast_convert_type(
        v[:, :m].astype(jnp.bfloat16), jnp.uint16).astype(jnp.uint32)
    hi = lax.bitcast_convert_type(
        v[:, m:].astype(jnp.bfloat16), jnp.uint16).astype(jnp.uint32)
    packed = lax.bitcast_convert_type((hi << 16) | lo, jnp.float32)
    return jnp.concatenate([packed, jnp.zeros_like(packed)], axis=1)


def _mm_body(x_ref, w_ref, o_ref):
    o_ref[...] = _pack_cols(jnp.dot(x_ref[...], w_ref[...],
                                    preferred_element_type=jnp.float32))


def _addrelu_body(p0_ref, p1_ref, o_ref):
    o_ref[...] = _pack_cols(jnp.maximum(p0_ref[0] + p1_ref[0], 0.0))


def _addmm_body(q0_ref, q1_ref, w_ref, o_ref):
    o_ref[...] = jnp.dot(q0_ref[0] + q1_ref[0], w_ref[...],
                         preferred_element_type=jnp.float32)


def _matmul(x, w, bm):
    n, kdim = x.shape
    kdim2, m = w.shape
    grid = n // bm
    return pl.pallas_call(
        _mm_body,
        grid=(grid,),
        in_specs=[pl.BlockSpec((bm, kdim), lambda i: (i, 0)),
                  pl.BlockSpec((kdim2, m), lambda i: (0, 0))],
        out_specs=pl.BlockSpec((bm, m), lambda i: (i, 0)),
        out_shape=jax.ShapeDtypeStruct((n, m), jnp.float32),
    )(x, w)


def _add_relu(p, bm):
    _, n, m = p.shape
    grid = n // bm
    return pl.pallas_call(
        _addrelu_body,
        grid=(grid,),
        in_specs=[pl.BlockSpec((1, bm, m), lambda i: (0, i, 0)),
                  pl.BlockSpec((1, bm, m), lambda i: (1, i, 0))],
        out_specs=pl.BlockSpec((bm, m), lambda i: (i, 0)),
        out_shape=jax.ShapeDtypeStruct((n, m), jnp.float32),
    )(p, p)


def _add_matmul(q, w, bm):
    _, n, kdim = q.shape
    kdim2, m = w.shape
    grid = n // bm
    return pl.pallas_call(
        _addmm_body,
        grid=(grid,),
        in_specs=[pl.BlockSpec((1, bm, kdim), lambda i: (0, i, 0)),
                  pl.BlockSpec((1, bm, kdim), lambda i: (1, i, 0)),
                  pl.BlockSpec((kdim2, m), lambda i: (0, 0))],
        out_specs=pl.BlockSpec((bm, m), lambda i: (i, 0)),
        out_shape=jax.ShapeDtypeStruct((n, m), jnp.float32),
    )(q, q, w)


def kernel(x, edge_index, edge_weight, W1, W2):
    n_nodes, f_in = x.shape
    n_edges = edge_index.shape[1]
    hidden = W1.shape[1]
    n_class = W2.shape[1]

    ei = edge_index.astype(jnp.int32).reshape(2 * n_edges)

    spmm = _make_spmm(n_nodes, n_edges, hidden)

    h1 = _matmul(x.astype(jnp.bfloat16), W1.astype(jnp.bfloat16), 1000)
    p = spmm(ei, edge_weight, h1)
    h = _add_relu(p, 1000)
    q = spmm(ei, edge_weight, h)
    return _add_matmul(q, W2, 1000)


# speedup vs baseline: 1.0099x; 1.0099x over previous
import functools

import jax
import jax.numpy as jnp
from jax import lax
from jax.experimental import pallas as pl
from jax.experimental.pallas import tpu as pltpu
from jax.experimental.pallas import tpu_sc as plsc

NUM_CORES = 2
NUM_SUBCORES = 16
NUM_WORKERS = NUM_CORES * NUM_SUBCORES
LANES = 16


def _make_spmm(n_nodes: int, n_edges: int, d: int):
    epw = n_edges // NUM_WORKERS
    K = 80
    ch = epw // K
    assert epw * NUM_WORKERS == n_edges and ch * K == epw
    nblk = n_nodes // K
    assert nblk * K == n_nodes
    bps = -(-nblk // NUM_SUBCORES)

    mesh = plsc.VectorSubcoreMesh(core_axis_name="c", subcore_axis_name="s")

    NB = 3
    UNROLL = 5
    assert K % UNROLL == 0
    assert ch >= 5 and (ch - 2) % NB == 0

    @functools.partial(
        pl.kernel,
        out_type=jax.ShapeDtypeStruct((NUM_CORES, n_nodes, d), jnp.float32),
        mesh=mesh,
        scratch_types=(
            [pltpu.VMEM((2, K), jnp.int32) for _ in range(NB)]
            + [pltpu.VMEM((K,), jnp.float32) for _ in range(NB)]
            + [pltpu.VMEM((K, d), jnp.float32) for _ in range(NB)]
            + [pltpu.VMEM_SHARED((n_nodes, d), jnp.float32)]
            + [pltpu.SemaphoreType.DMA for _ in range(3 * NB)]
        ),
        compiler_params=pltpu.CompilerParams(needs_layout_passes=False),
    )
    def spmm(ei_hbm, w_hbm, h_hbm, out_hbm,
             sd0, sd1, sd2, wv0, wv1, wv2, rv0, rv1, rv2,
             acc, as0, as1, as2, gs0, gs1, gs2, ss0, ss1, ss2):
        sd_v, w_v = (sd0, sd1, sd2), (wv0, wv1, wv2)
        rows_v = (rv0, rv1, rv2)
        asem, gsem, ssem = (as0, as1, as2), (gs0, gs1, gs2), (ss0, ss1, ss2)

        c = lax.axis_index("c")
        s = lax.axis_index("s")
        wid = c * NUM_SUBCORES + s
        ebase = wid * epw

        def stage(i, b):
            ebeg = ebase + i * K
            pltpu.async_copy(
                ei_hbm.at[pl.ds(ebeg, K)], sd_v[b].at[0], asem[b])
            pltpu.async_copy(
                ei_hbm.at[pl.ds(n_edges + ebeg, K)], sd_v[b].at[1], asem[b])
            pltpu.async_copy(w_hbm.at[pl.ds(ebeg, K)], w_v[b], asem[b])

        def wait_stage(b):
            dsl = pl.ds(0, K)
            pltpu.make_async_copy(ei_hbm.at[dsl], sd_v[b].at[0], asem[b]).wait()
            pltpu.make_async_copy(ei_hbm.at[dsl], sd_v[b].at[1], asem[b]).wait()
            pltpu.make_async_copy(w_hbm.at[dsl], w_v[b], asem[b]).wait()

        def gather(b):
            pltpu.async_copy(h_hbm.at[sd_v[b].at[0]], rows_v[b], gsem[b])

        def wait_gather(b):
            pltpu.make_async_copy(
                h_hbm.at[pl.ds(0, K)], rows_v[b], gsem[b]).wait()

        def scale(b):
            @plsc.parallel_loop(0, K, 1, unroll=UNROLL)
            def _(j):
                wvec = plsc.load_gather(
                    w_v[b], [jnp.full((LANES,), j, jnp.int32)])
                packs = [
                    plsc.bitcast(rows_v[b][j, pl.ds(v * LANES, LANES)],
                                 jnp.bfloat16)
                    for v in range(d // (2 * LANES))
                ]
                pairs = [plsc.unpack(p, format=plsc.PackFormat.INTERLEAVED)
                         for p in packs]
                for v, (lo, hi) in enumerate(pairs):
                    rows_v[b][j, pl.ds(v * LANES, LANES)] = lo * wvec
                for v, (lo, hi) in enumerate(pairs):
                    rows_v[b][j, pl.ds(d // 2 + v * LANES, LANES)] = hi * wvec

        def scatter(b):
            pltpu.async_copy(rows_v[b], acc.at[sd_v[b].at[1]], ssem[b],
                             add=True)

        def wait_scatter(b):
            pltpu.make_async_copy(
                rows_v[b], acc.at[pl.ds(0, K)], ssem[b]).wait()

        def zrow(r, carry):
            for v in range(d // LANES):
                rv0[r, pl.ds(v * LANES, LANES)] = jnp.zeros(
                    (LANES,), jnp.float32)
            return carry
        lax.fori_loop(0, K, zrow, 0)
        for k in range(bps):
            blk = s + NUM_SUBCORES * k

            @pl.when(blk < nblk)
            def _():
                pltpu.sync_copy(rv0, acc.at[pl.ds(blk * K, K)])
        plsc.subcore_barrier()

        stage(0, 0)
        stage(1, 1)
        wait_stage(0)
        gather(0)
        wait_stage(1)
        gather(1)
        wait_gather(0)
        scale(0)
        scatter(0)
        stage(2, 2)
        wait_stage(2)
        gather(2)
        wait_gather(1)
        scale(1)
        scatter(1)
        wait_scatter(0)
        stage(3, 0)

        def body(t, carry):
            i0 = 3 * t + 2
            for k in range(3):
                ph = (2 + k) % 3
                nxt = (ph + 1) % 3
                prv = (ph + 2) % 3
                i = i0 + k
                wait_stage(nxt)
                gather(nxt)
                wait_gather(ph)
                scale(ph)
                scatter(ph)
                wait_scatter(prv)
                stage(i + 2, prv)
            return carry
        lax.fori_loop(0, (ch - 5) // 3, body, 0)

        wait_stage(0)
        gather(0)
        wait_gather(2)
        scale(2)
        scatter(2)
        wait_scatter(1)
        stage(ch - 1, 1)
        wait_stage(1)
        gather(1)
        wait_gather(0)
        scale(0)
        scatter(0)
        wait_gather(1)
        scale(1)
        scatter(1)
        wait_scatter(2)
        wait_scatter(0)
        wait_scatter(1)
        plsc.subcore_barrier()

        for k in range(bps):
            blk = s + NUM_SUBCORES * k

            @pl.when(blk < nblk)
            def _():
                sl = pl.ds(blk * K, K)
                pltpu.sync_copy(acc.at[sl], out_hbm.at[c, sl])

    return spmm


def _pack2(lo, hi):
    lob = lax.bitcast_convert_type(
        lo.astype(jnp.bfloat16), jnp.uint16).astype(jnp.uint32)
    hib = lax.bitcast_convert_type(
        hi.astype(jnp.bfloat16), jnp.uint16).astype(jnp.uint32)
    return lax.bitcast_convert_type((hib << 16) | lob, jnp.float32)


def _pack_cols(v):
    m = v.shape[1] // 2
    packed = _pack2(v[:, :m], v[:, m:])
    return jnp.concatenate([packed, jnp.zeros_like(packed)], axis=1)


def _mm_body(x_ref, w_ref, o_ref):
    o_ref[...] = _pack_cols(
        jnp.dot(x_ref[...].astype(jnp.bfloat16),
                w_ref[...].astype(jnp.bfloat16),
                preferred_element_type=jnp.float32))


def _addrelu_body(p0_ref, p1_ref, o_ref):
    o_ref[...] = _pack_cols(jnp.maximum(p0_ref[0] + p1_ref[0], 0.0))


def _addmm_body(q0_ref, q1_ref, w_ref, o_ref):
    o_ref[...] = jnp.dot(q0_ref[0] + q1_ref[0], w_ref[...],
                         preferred_element_type=jnp.float32)


def _matmul(x, w, bm):
    n, kdim = x.shape
    kdim2, m = w.shape
    grid = n // bm
    return pl.pallas_call(
        _mm_body,
        grid=(grid,),
        in_specs=[pl.BlockSpec((bm, kdim), lambda i: (i, 0)),
                  pl.BlockSpec((kdim2, m), lambda i: (0, 0))],
        out_specs=pl.BlockSpec((bm, m), lambda i: (i, 0)),
        out_shape=jax.ShapeDtypeStruct((n, m), jnp.float32),
    )(x, w)


def _add_relu(p, bm):
    _, n, m = p.shape
    grid = n // bm
    return pl.pallas_call(
        _addrelu_body,
        grid=(grid,),
        in_specs=[pl.BlockSpec((1, bm, m), lambda i: (0, i, 0)),
                  pl.BlockSpec((1, bm, m), lambda i: (1, i, 0))],
        out_specs=pl.BlockSpec((bm, m), lambda i: (i, 0)),
        out_shape=jax.ShapeDtypeStruct((n, m), jnp.float32),
    )(p, p)


def _add_matmul(q, w, bm):
    _, n, kdim = q.shape
    kdim2, m = w.shape
    grid = n // bm
    return pl.pallas_call(
        _addmm_body,
        grid=(grid,),
        in_specs=[pl.BlockSpec((1, bm, kdim), lambda i: (0, i, 0)),
                  pl.BlockSpec((1, bm, kdim), lambda i: (1, i, 0)),
                  pl.BlockSpec((kdim2, m), lambda i: (0, 0))],
        out_specs=pl.BlockSpec((bm, m), lambda i: (i, 0)),
        out_shape=jax.ShapeDtypeStruct((n, m), jnp.float32),
    )(q, q, w)


def kernel(x, edge_index, edge_weight, W1, W2):
    n_nodes, f_in = x.shape
    n_edges = edge_index.shape[1]
    hidden = W1.shape[1]

    ei = edge_index.astype(jnp.int32).reshape(2 * n_edges)

    spmm = _make_spmm(n_nodes, n_edges, hidden)

    h1 = _matmul(x, W1, 1000)
    p = spmm(ei, edge_weight, h1)
    h = _add_relu(p, 1000)
    q = spmm(ei, edge_weight, h)
    return _add_matmul(q, W2, 1000)


# speedup vs baseline: 1.0122x; 1.0023x over previous
import functools

import jax
import jax.numpy as jnp
from jax import lax
from jax.experimental import pallas as pl
from jax.experimental.pallas import tpu as pltpu
from jax.experimental.pallas import tpu_sc as plsc

NUM_CORES = 2
NUM_SUBCORES = 16
NUM_WORKERS = NUM_CORES * NUM_SUBCORES
LANES = 16


def _make_spmm(n_nodes: int, n_edges: int, d: int):
    epw = n_edges // NUM_WORKERS
    K = 80
    ch = epw // K
    assert epw * NUM_WORKERS == n_edges and ch * K == epw
    nblk = n_nodes // K
    assert nblk * K == n_nodes
    bps = -(-nblk // NUM_SUBCORES)

    mesh = plsc.VectorSubcoreMesh(core_axis_name="c", subcore_axis_name="s")

    NB = 3
    UNROLL = 4
    assert K % UNROLL == 0
    assert ch >= 5 and (ch - 2) % NB == 0

    @functools.partial(
        pl.kernel,
        out_type=jax.ShapeDtypeStruct((NUM_CORES, n_nodes, d), jnp.float32),
        mesh=mesh,
        scratch_types=(
            [pltpu.VMEM((2, K), jnp.int32) for _ in range(NB)]
            + [pltpu.VMEM((K,), jnp.float32) for _ in range(NB)]
            + [pltpu.VMEM((K, d), jnp.float32) for _ in range(NB)]
            + [pltpu.VMEM_SHARED((n_nodes, d), jnp.float32)]
            + [pltpu.SemaphoreType.DMA for _ in range(3 * NB)]
        ),
        compiler_params=pltpu.CompilerParams(needs_layout_passes=False),
    )
    def spmm(ei_hbm, w_hbm, h_hbm, out_hbm,
             sd0, sd1, sd2, wv0, wv1, wv2, rv0, rv1, rv2,
             acc, as0, as1, as2, gs0, gs1, gs2, ss0, ss1, ss2):
        sd_v, w_v = (sd0, sd1, sd2), (wv0, wv1, wv2)
        rows_v = (rv0, rv1, rv2)
        asem, gsem, ssem = (as0, as1, as2), (gs0, gs1, gs2), (ss0, ss1, ss2)

        c = lax.axis_index("c")
        s = lax.axis_index("s")
        wid = c * NUM_SUBCORES + s
        ebase = wid * epw

        def stage(i, b):
            ebeg = ebase + i * K
            pltpu.async_copy(
                ei_hbm.at[pl.ds(ebeg, K)], sd_v[b].at[0], asem[b])
            pltpu.async_copy(
                ei_hbm.at[pl.ds(n_edges + ebeg, K)], sd_v[b].at[1], asem[b])
            pltpu.async_copy(w_hbm.at[pl.ds(ebeg, K)], w_v[b], asem[b])

        def wait_stage(b):
            dsl = pl.ds(0, K)
            pltpu.make_async_copy(ei_hbm.at[dsl], sd_v[b].at[0], asem[b]).wait()
            pltpu.make_async_copy(ei_hbm.at[dsl], sd_v[b].at[1], asem[b]).wait()
            pltpu.make_async_copy(w_hbm.at[dsl], w_v[b], asem[b]).wait()

        def gather(b):
            pltpu.async_copy(h_hbm.at[sd_v[b].at[0]], rows_v[b], gsem[b])

        def wait_gather(b):
            pltpu.make_async_copy(
                h_hbm.at[pl.ds(0, K)], rows_v[b], gsem[b]).wait()

        def scale(b):
            @plsc.parallel_loop(0, K, 1, unroll=UNROLL)
            def _(j):
                wvec = plsc.load_gather(
                    w_v[b], [jnp.full((LANES,), j, jnp.int32)])
                packs = [
                    plsc.bitcast(rows_v[b][j, pl.ds(v * LANES, LANES)],
                                 jnp.bfloat16)
                    for v in range(d // (2 * LANES))
                ]
                pairs = [plsc.unpack(p, format=plsc.PackFormat.INTERLEAVED)
                         for p in packs]
                for v, (lo, hi) in enumerate(pairs):
                    rows_v[b][j, pl.ds(v * LANES, LANES)] = lo * wvec
                for v, (lo, hi) in enumerate(pairs):
                    rows_v[b][j, pl.ds(d // 2 + v * LANES, LANES)] = hi * wvec

        def scatter(b):
            pltpu.async_copy(rows_v[b], acc.at[sd_v[b].at[1]], ssem[b],
                             add=True)

        def wait_scatter(b):
            pltpu.make_async_copy(
                rows_v[b], acc.at[pl.ds(0, K)], ssem[b]).wait()

        def zrow(r, carry):
            for v in range(d // LANES):
                rv0[r, pl.ds(v * LANES, LANES)] = jnp.zeros(
                    (LANES,), jnp.float32)
            return carry
        lax.fori_loop(0, K, zrow, 0)
        for k in range(bps):
            blk = s + NUM_SUBCORES * k

            @pl.when(blk < nblk)
            def _():
                pltpu.sync_copy(rv0, acc.at[pl.ds(blk * K, K)])
        plsc.subcore_barrier()

        stage(0, 0)
        stage(1, 1)
        wait_stage(0)
        gather(0)
        wait_stage(1)
        gather(1)
        wait_gather(0)
        scale(0)
        scatter(0)
        stage(2, 2)
        wait_stage(2)
        gather(2)
        wait_gather(1)
        scale(1)
        scatter(1)
        wait_scatter(0)
        stage(3, 0)

        def body(t, carry):
            i0 = 3 * t + 2
            for k in range(3):
                ph = (2 + k) % 3
                nxt = (ph + 1) % 3
                prv = (ph + 2) % 3
                i = i0 + k
                wait_stage(nxt)
                gather(nxt)
                wait_gather(ph)
                scale(ph)
                scatter(ph)
                wait_scatter(prv)
                stage(i + 2, prv)
            return carry
        lax.fori_loop(0, (ch - 5) // 3, body, 0)

        wait_stage(0)
        gather(0)
        wait_gather(2)
        scale(2)
        scatter(2)
        wait_scatter(1)
        stage(ch - 1, 1)
        wait_stage(1)
        gather(1)
        wait_gather(0)
        scale(0)
        scatter(0)
        wait_gather(1)
        scale(1)
        scatter(1)
        wait_scatter(2)
        wait_scatter(0)
        wait_scatter(1)
        plsc.subcore_barrier()

        for k in range(bps):
            blk = s + NUM_SUBCORES * k

            @pl.when(blk < nblk)
            def _():
                sl = pl.ds(blk * K, K)
                pltpu.sync_copy(acc.at[sl], out_hbm.at[c, sl])

    return spmm


def _pack2(lo, hi):
    lob = lax.bitcast_convert_type(
        lo.astype(jnp.bfloat16), jnp.uint16).astype(jnp.uint32)
    hib = lax.bitcast_convert_type(
        hi.astype(jnp.bfloat16), jnp.uint16).astype(jnp.uint32)
    return lax.bitcast_convert_type((hib << 16) | lob, jnp.float32)


def _pack_cols(v):
    m = v.shape[1] // 2
    packed = _pack2(v[:, :m], v[:, m:])
    return jnp.concatenate([packed, jnp.zeros_like(packed)], axis=1)


def _mm_body(x_ref, w_ref, o_ref):
    o_ref[...] = _pack_cols(
        jnp.dot(x_ref[...].astype(jnp.bfloat16),
                w_ref[...].astype(jnp.bfloat16),
                preferred_element_type=jnp.float32))


def _addrelu_body(p0_ref, p1_ref, o_ref):
    o_ref[...] = _pack_cols(jnp.maximum(p0_ref[0] + p1_ref[0], 0.0))


def _addmm_body(q0_ref, q1_ref, w_ref, o_ref):
    o_ref[...] = jnp.dot(q0_ref[0] + q1_ref[0], w_ref[...],
                         preferred_element_type=jnp.float32)


def _matmul(x, w, bm):
    n, kdim = x.shape
    kdim2, m = w.shape
    grid = n // bm
    return pl.pallas_call(
        _mm_body,
        grid=(grid,),
        in_specs=[pl.BlockSpec((bm, kdim), lambda i: (i, 0)),
                  pl.BlockSpec((kdim2, m), lambda i: (0, 0))],
        out_specs=pl.BlockSpec((bm, m), lambda i: (i, 0)),
        out_shape=jax.ShapeDtypeStruct((n, m), jnp.float32),
    )(x, w)


def _add_relu(p, bm):
    _, n, m = p.shape
    grid = n // bm
    return pl.pallas_call(
        _addrelu_body,
        grid=(grid,),
        in_specs=[pl.BlockSpec((1, bm, m), lambda i: (0, i, 0)),
                  pl.BlockSpec((1, bm, m), lambda i: (1, i, 0))],
        out_specs=pl.BlockSpec((bm, m), lambda i: (i, 0)),
        out_shape=jax.ShapeDtypeStruct((n, m), jnp.float32),
    )(p, p)


def _add_matmul(q, w, bm):
    _, n, kdim = q.shape
    kdim2, m = w.shape
    grid = n // bm
    return pl.pallas_call(
        _addmm_body,
        grid=(grid,),
        in_specs=[pl.BlockSpec((1, bm, kdim), lambda i: (0, i, 0)),
                  pl.BlockSpec((1, bm, kdim), lambda i: (1, i, 0)),
                  pl.BlockSpec((kdim2, m), lambda i: (0, 0))],
        out_specs=pl.BlockSpec((bm, m), lambda i: (i, 0)),
        out_shape=jax.ShapeDtypeStruct((n, m), jnp.float32),
    )(q, q, w)


def kernel(x, edge_index, edge_weight, W1, W2):
    n_nodes, f_in = x.shape
    n_edges = edge_index.shape[1]
    hidden = W1.shape[1]

    ei = edge_index.astype(jnp.int32).reshape(2 * n_edges)

    spmm = _make_spmm(n_nodes, n_edges, hidden)

    h1 = _matmul(x, W1, 1000)
    p = spmm(ei, edge_weight, h1)
    h = _add_relu(p, 1000)
    q = spmm(ei, edge_weight, h)
    return _add_matmul(q, W2, 1000)
